# Initial kernel scaffold; baseline (speedup 1.0000x reference)
import jax
import jax.numpy as jnp
from jax.experimental import pallas as pl


def kernel(x, label_mask, global_prototype_tensor, W_hidden, b_hidden, W_proto, b_proto):
    raise NotImplementedError("write your pallas kernel here")


# speedup vs baseline: 1.9072x; 1.9072x over previous
import functools

import jax
import jax.numpy as jnp
from jax import lax
from jax.experimental import pallas as pl
from jax.experimental.pallas import tpu as pltpu

_B = 256
_L = 1000
_D = 128
_H = 256
_NB = 8
_STEPS = _B // _NB


def _mlp(v, wh, bh, wp, bp):
    h = jnp.maximum(
        lax.dot_general(v, wh, (((1,), (1,)), ((), ())),
                        preferred_element_type=jnp.float32,
                        precision=lax.Precision.HIGHEST) + bh,
        0.0)
    return lax.dot_general(h, wp, (((1,), (1,)), ((), ())),
                           preferred_element_type=jnp.float32,
                           precision=lax.Precision.HIGHEST) + bp


def _body(x_ref, mask_ref, gpt_ref, gpt_last_ref, wh_ref, bh_ref, wp_ref,
          bp_ref, proto_ref, anti_ref, pos_acc, tot_acc, cnt_acc):
    i = pl.program_id(0)

    @pl.when(i == 0)
    def _init():
        pos_acc[...] = jnp.zeros_like(pos_acc)
        tot_acc[...] = jnp.zeros_like(tot_acc)
        cnt_acc[...] = jnp.zeros_like(cnt_acc)

    xb = x_ref[...]
    mb = mask_ref[...].astype(jnp.float32)
    pos_acc[...] += jnp.sum(mb[:, :, None] * xb, axis=0)
    tot_acc[...] += jnp.sum(xb, axis=0)
    cnt_acc[...] += lax.dot_general(
        mb, jnp.ones((_NB, 1), jnp.float32), (((0,), (0,)), ((), ())),
        preferred_element_type=jnp.float32,
        precision=lax.Precision.HIGHEST)

    @pl.when(i == _STEPS - 1)
    def _finish():
        cnt = cnt_acc[...]
        pos = pos_acc[...]
        tot = tot_acc[...]
        neg_cnt = _B - cnt
        pos_mean = pos / jnp.maximum(cnt, 1.0)
        neg_mean = (tot - pos) / jnp.maximum(neg_cnt, 1.0)
        avg = 0.5 * pos_mean + 0.5 * gpt_ref[...]
        avg_anti = 0.5 * neg_mean + 0.5 * gpt_last_ref[...]
        wh = wh_ref[...]
        bh = bh_ref[...]
        wp = wp_ref[...]
        bp = bp_ref[...]
        proto = _mlp(avg, wh, bh, wp, bp)
        proto = jnp.where(cnt > 0.0, proto, 0.0)
        anti = _mlp(avg_anti, wh, bh, wp, bp)
        valid = (neg_cnt > 0.0).astype(jnp.float32)
        anti_sum = jnp.sum(anti * valid, axis=0, keepdims=True)
        anti_row = anti_sum / jnp.maximum(jnp.sum(valid), 1.0)
        proto_ref[...] = proto
        anti_ref[...] = anti_row


@jax.jit
def _run(x, label_mask, gpt_main, gpt_last, wh, bh, wp, bp):
    grid = (_STEPS,)
    proto, anti = pl.pallas_call(
        _body,
        grid=grid,
        in_specs=[
            pl.BlockSpec((_NB, _L, _D), lambda i: (i, 0, 0)),
            pl.BlockSpec((_NB, _L), lambda i: (i, 0)),
            pl.BlockSpec((_L, _D), lambda i: (0, 0)),
            pl.BlockSpec((1, _D), lambda i: (0, 0)),
            pl.BlockSpec((_H, _D), lambda i: (0, 0)),
            pl.BlockSpec((1, _H), lambda i: (0, 0)),
            pl.BlockSpec((_D, _H), lambda i: (0, 0)),
            pl.BlockSpec((1, _D), lambda i: (0, 0)),
        ],
        out_specs=[
            pl.BlockSpec((_L, _D), lambda i: (0, 0)),
            pl.BlockSpec((1, _D), lambda i: (0, 0)),
        ],
        out_shape=[
            jax.ShapeDtypeStruct((_L, _D), jnp.float32),
            jax.ShapeDtypeStruct((1, _D), jnp.float32),
        ],
        scratch_shapes=[
            pltpu.VMEM((_L, _D), jnp.float32),
            pltpu.VMEM((_L, _D), jnp.float32),
            pltpu.VMEM((_L, 1), jnp.float32),
        ],
        compiler_params=pltpu.CompilerParams(
            dimension_semantics=("arbitrary",),
        ),
    )(x, label_mask, gpt_main, gpt_last, wh, bh, wp, bp)
    return jnp.concatenate([proto, anti], axis=0)


def kernel(x, label_mask, global_prototype_tensor, W_hidden, b_hidden,
           W_proto, b_proto):
    gpt_main = global_prototype_tensor[:_L]
    gpt_last = global_prototype_tensor[_L:]
    return _run(x, label_mask, gpt_main, gpt_last, W_hidden,
                b_hidden.reshape(1, _H), W_proto, b_proto.reshape(1, _D))
